# baseline (speedup 1.0000x reference)
import functools

import jax
import jax.numpy as jnp
from jax import lax
from jax.experimental import pallas as pl
from jax.experimental.pallas import tpu as pltpu
from jax.experimental.pallas import tpu_sc as plsc

_NC = 2
_NS = 16
_NW = _NC * _NS
_BLK = 128


def _make_gather(n_idx, dim):
    nblk = n_idx // _BLK
    blk_per_w = nblk // _NW
    mesh = plsc.VectorSubcoreMesh(core_axis_name="c", subcore_axis_name="s")

    @functools.partial(
        pl.kernel,
        mesh=mesh,
        out_type=jax.ShapeDtypeStruct((n_idx, dim), jnp.float32),
        scratch_types=[
            pltpu.VMEM((blk_per_w, _BLK), jnp.int32),
            pltpu.VMEM((_BLK, dim), jnp.float32),
            pltpu.SemaphoreType.DMA,
        ],
        compiler_params=pltpu.CompilerParams(use_tc_tiling_on_sc=False),
    )
    def k(idx_hbm, table_hbm, out_hbm, idx_v, rows_v, gsem):
        wid = lax.axis_index("s") * _NC + lax.axis_index("c")
        blk0 = wid * blk_per_w
        pltpu.sync_copy(idx_hbm.at[pl.ds(blk0, blk_per_w)], idx_v)

        def body(j, carry):
            pltpu.async_copy(table_hbm.at[idx_v.at[j]], rows_v, gsem).wait()
            pltpu.sync_copy(rows_v, out_hbm.at[pl.ds((blk0 + j) * _BLK, _BLK)])
            return carry

        lax.fori_loop(0, blk_per_w, body, 0)

    return k


def kernel(indices, table):
    b, l = indices.shape
    n = b * l
    dim = table.shape[1]
    flat = indices.reshape(n // _BLK, _BLK).astype(jnp.int32)
    out = _make_gather(n, dim)(flat, table)
    return out.reshape(b, l, dim)


# speedup vs baseline: 1.1148x; 1.1148x over previous
import functools

import jax
import jax.numpy as jnp
from jax import lax
from jax.experimental import pallas as pl
from jax.experimental.pallas import tpu as pltpu
from jax.experimental.pallas import tpu_sc as plsc

_NC = 2
_NS = 16
_NW = _NC * _NS
_BLK = 128
_NBUF = 10


def _make_gather(n_idx, dim):
    nblk = n_idx // _BLK
    blk_per_w = nblk // _NW
    n_groups = blk_per_w // _NBUF
    mesh = plsc.VectorSubcoreMesh(core_axis_name="c", subcore_axis_name="s")

    @functools.partial(
        pl.kernel,
        mesh=mesh,
        out_type=jax.ShapeDtypeStruct((n_idx, dim), jnp.float32),
        scratch_types=[
            pltpu.VMEM((blk_per_w, _BLK), jnp.int32),
            pltpu.VMEM((_NBUF, _BLK, dim), jnp.float32),
            pltpu.SemaphoreType.DMA((_NBUF,)),
            pltpu.SemaphoreType.DMA((_NBUF,)),
        ],
        compiler_params=pltpu.CompilerParams(use_tc_tiling_on_sc=False),
    )
    def k(idx_hbm, table_hbm, out_hbm, idx_v, bufs, gsem, wsem):
        wid = lax.axis_index("s") * _NC + lax.axis_index("c")
        blk0 = wid * blk_per_w
        pltpu.sync_copy(idx_hbm.at[pl.ds(blk0, blk_per_w)], idx_v)

        def gather_copy(j, b):
            return pltpu.make_async_copy(
                table_hbm.at[idx_v.at[j]], bufs.at[b], gsem.at[b])

        def write_copy(j, b):
            return pltpu.make_async_copy(
                bufs.at[b], out_hbm.at[pl.ds((blk0 + j) * _BLK, _BLK)],
                wsem.at[b])

        for b in range(_NBUF):
            gather_copy(b, b).start()

        def group(g, carry):
            j0 = g * _NBUF
            for b in range(_NBUF):
                j = j0 + b
                gather_copy(j, b).wait()
                write_copy(j, b).start()
                write_copy(j, b).wait()
                gather_copy(j + _NBUF, b).start()
            return carry

        lax.fori_loop(0, n_groups - 1, group, 0)

        j0 = (n_groups - 1) * _NBUF
        for b in range(_NBUF):
            j = j0 + b
            gather_copy(j, b).wait()
            write_copy(j, b).start()
            write_copy(j, b).wait()

    return k


def kernel(indices, table):
    b, l = indices.shape
    n = b * l
    dim = table.shape[1]
    flat = indices.reshape(n // _BLK, _BLK).astype(jnp.int32)
    out = _make_gather(n, dim)(flat, table)
    return out.reshape(b, l, dim)


# speedup vs baseline: 1.1160x; 1.0011x over previous
import functools

import jax
import jax.numpy as jnp
from jax import lax
from jax.experimental import pallas as pl
from jax.experimental.pallas import tpu as pltpu
from jax.experimental.pallas import tpu_sc as plsc

_NC = 2
_NS = 16
_NW = _NC * _NS
_NBUF = 8


def _make_gather(b, l, dim):
    rows_per_w = b // _NW
    n_groups = rows_per_w // _NBUF
    mesh = plsc.VectorSubcoreMesh(core_axis_name="c", subcore_axis_name="s")

    @functools.partial(
        pl.kernel,
        mesh=mesh,
        out_type=jax.ShapeDtypeStruct((b, l, dim), jnp.float32),
        scratch_types=[
            pltpu.VMEM((rows_per_w, l), jnp.int32),
            pltpu.VMEM((_NBUF, l, dim), jnp.float32),
            pltpu.SemaphoreType.DMA((_NBUF,)),
            pltpu.SemaphoreType.DMA((_NBUF,)),
        ],
        compiler_params=pltpu.CompilerParams(use_tc_tiling_on_sc=False),
    )
    def k(idx_hbm, table_hbm, out_hbm, idx_v, bufs, gsem, wsem):
        wid = lax.axis_index("s") * _NC + lax.axis_index("c")
        r0 = wid * rows_per_w
        pltpu.sync_copy(idx_hbm.at[pl.ds(r0, rows_per_w)], idx_v)

        def gather_copy(r, s):
            return pltpu.make_async_copy(
                table_hbm.at[idx_v.at[r]], bufs.at[s], gsem.at[s])

        def write_copy(r, s):
            return pltpu.make_async_copy(
                bufs.at[pl.ds(s, 1)], out_hbm.at[pl.ds(r0 + r, 1)],
                wsem.at[s])

        for s in range(_NBUF):
            gather_copy(s, s).start()

        def group(g, carry):
            row0 = g * _NBUF
            for s in range(_NBUF):
                r = row0 + s
                gather_copy(r, s).wait()
                write_copy(r, s).start()
                write_copy(r, s).wait()
                gather_copy(r + _NBUF, s).start()
            return carry

        lax.fori_loop(0, n_groups - 1, group, 0)

        row0 = (n_groups - 1) * _NBUF
        for s in range(_NBUF):
            r = row0 + s
            gather_copy(r, s).wait()
            write_copy(r, s).start()
            write_copy(r, s).wait()

    return k


def kernel(indices, table):
    b, l = indices.shape
    dim = table.shape[1]
    return _make_gather(b, l, dim)(indices.astype(jnp.int32), table)


# speedup vs baseline: 1.4721x; 1.3191x over previous
import functools

import jax
import jax.numpy as jnp
from jax import lax
from jax.experimental import pallas as pl
from jax.experimental.pallas import tpu as pltpu
from jax.experimental.pallas import tpu_sc as plsc

_NC = 2
_NS = 16
_NW = _NC * _NS
_NBUF = 4
_PD = 128


def _make_gather(b, l, dim):
    rows_per_w = b // _NW
    n_groups = rows_per_w // _NBUF
    mesh = plsc.VectorSubcoreMesh(core_axis_name="c", subcore_axis_name="s")

    @functools.partial(
        pl.kernel,
        mesh=mesh,
        out_type=jax.ShapeDtypeStruct((b, l, _PD), jnp.float32),
        scratch_types=[
            pltpu.VMEM((rows_per_w, l), jnp.int32),
            pltpu.VMEM((_NBUF, l, _PD), jnp.float32),
            pltpu.SemaphoreType.DMA((_NBUF,)),
            pltpu.SemaphoreType.DMA((_NBUF,)),
        ],
        compiler_params=pltpu.CompilerParams(use_tc_tiling_on_sc=False),
    )
    def k(idx_hbm, table_hbm, out_hbm, idx_v, bufs, gsem, wsem):
        wid = lax.axis_index("s") * _NC + lax.axis_index("c")
        r0 = wid * rows_per_w
        pltpu.sync_copy(idx_hbm.at[pl.ds(r0, rows_per_w)], idx_v)

        def gather_copy(r, s):
            return pltpu.make_async_copy(
                table_hbm.at[idx_v.at[r]], bufs.at[s], gsem.at[s])

        def write_copy(r, s):
            return pltpu.make_async_copy(
                bufs.at[pl.ds(s, 1), :, pl.ds(0, dim)],
                out_hbm.at[pl.ds(r0 + r, 1), :, pl.ds(0, dim)],
                wsem.at[s])

        for s in range(_NBUF):
            gather_copy(s, s).start()

        def group(g, carry):
            row0 = g * _NBUF
            for s in range(_NBUF):
                r = row0 + s
                gather_copy(r, s).wait()
                write_copy(r, s).start()
                write_copy(r, s).wait()
                gather_copy(r + _NBUF, s).start()
            return carry

        lax.fori_loop(0, n_groups - 1, group, 0)

        row0 = (n_groups - 1) * _NBUF
        for s in range(_NBUF):
            r = row0 + s
            gather_copy(r, s).wait()
            write_copy(r, s).start()
            write_copy(r, s).wait()

    return k


def kernel(indices, table):
    b, l = indices.shape
    v, dim = table.shape
    padded = jnp.pad(table, ((0, 0), (0, _PD - dim)))
    res = _make_gather(b, l, dim)(indices.astype(jnp.int32), padded)
    return res[:, :, :dim]


# speedup vs baseline: 1.5905x; 1.0804x over previous
import functools

import jax
import jax.numpy as jnp
from jax import lax
from jax.experimental import pallas as pl
from jax.experimental.pallas import tpu as pltpu
from jax.experimental.pallas import tpu_sc as plsc

_NC = 2
_NS = 16
_NW = _NC * _NS
_NBUF = 8
_PD = 128


def _make_gather(b, l, dim):
    rows_per_w = b // _NW
    n_groups = rows_per_w // _NBUF
    mesh = plsc.VectorSubcoreMesh(core_axis_name="c", subcore_axis_name="s")

    @functools.partial(
        pl.kernel,
        mesh=mesh,
        out_type=jax.ShapeDtypeStruct((b, l, _PD), jnp.float32),
        scratch_types=[
            pltpu.VMEM((rows_per_w, l), jnp.int32),
            pltpu.VMEM((_NBUF, l, dim), jnp.float32),
            pltpu.SemaphoreType.DMA((_NBUF,)),
            pltpu.SemaphoreType.DMA((_NBUF,)),
        ],
        compiler_params=pltpu.CompilerParams(use_tc_tiling_on_sc=False),
    )
    def k(idx_hbm, table_hbm, out_hbm, idx_v, bufs, gsem, wsem):
        wid = lax.axis_index("s") * _NC + lax.axis_index("c")
        r0 = wid * rows_per_w
        pltpu.sync_copy(idx_hbm.at[pl.ds(r0, rows_per_w)], idx_v)

        def gather_copy(r, s):
            return pltpu.make_async_copy(
                table_hbm.at[idx_v.at[r]], bufs.at[s], gsem.at[s])

        def write_copy(r, s):
            return pltpu.make_async_copy(
                bufs.at[pl.ds(s, 1)],
                out_hbm.at[pl.ds(r0 + r, 1), :, pl.ds(0, dim)],
                wsem.at[s])

        for s in range(_NBUF):
            gather_copy(s, s).start()

        def group(g, carry):
            row0 = g * _NBUF
            for s in range(_NBUF):
                r = row0 + s
                gather_copy(r, s).wait()
                write_copy(r, s).start()
                write_copy(r, s).wait()
                gather_copy(r + _NBUF, s).start()
            return carry

        lax.fori_loop(0, n_groups - 1, group, 0)

        row0 = (n_groups - 1) * _NBUF
        for s in range(_NBUF):
            r = row0 + s
            gather_copy(r, s).wait()
            write_copy(r, s).start()
            write_copy(r, s).wait()

    return k


def kernel(indices, table):
    b, l = indices.shape
    v, dim = table.shape
    padded = jnp.pad(table, ((0, 0), (0, _PD - dim)))
    flat = padded.reshape(v * (_PD // dim), dim)
    idx2 = indices.astype(jnp.int32) * (_PD // dim)
    res = _make_gather(b, l, dim)(idx2, flat)
    return res[:, :, :dim]
